# baseline (speedup 1.0000x reference)
import jax
import jax.numpy as jnp
from jax import lax
from jax.experimental import pallas as pl
from jax.experimental.pallas import tpu as pltpu

_INV_SQRT2 = 0.7071067811865476

_ERF_A = (0.254829592, -0.284496736, 1.421413741, -1.453152027, 1.061405429)
_ERF_P = 0.3275911


def _erf_approx(v):
    a1, a2, a3, a4, a5 = _ERF_A
    s = jnp.where(v < 0.0, -1.0, 1.0)
    av = jnp.abs(v)
    t = 1.0 / (1.0 + _ERF_P * av)
    poly = t * (a1 + t * (a2 + t * (a3 + t * (a4 + t * a5))))
    return s * (1.0 - poly * jnp.exp(-av * av))


def _gelu(v):
    return 0.5 * v * (1.0 + _erf_approx(v * _INV_SQRT2))


def _se_nhwc_kernel(xa_ref, xb_ref, w1_ref, w2t_ref, o_ref):
    nb = xa_ref.shape[0]
    hw = xa_ref.shape[1] + xb_ref.shape[1]
    pooled = jnp.concatenate(
        [jnp.sum(xa_ref[i], axis=0, keepdims=True)
         + jnp.sum(xb_ref[i], axis=0, keepdims=True) for i in range(nb)],
        axis=0,
    ) * (1.0 / hw)
    h = lax.dot_general(pooled, w1_ref[...],
                        (((1,), (1,)), ((), ())),
                        preferred_element_type=jnp.float32)
    h = _gelu(h)
    g = lax.dot_general(h, w2t_ref[...],
                        (((1,), (0,)), ((), ())),
                        preferred_element_type=jnp.float32)
    gate = 1.0 / (1.0 + jnp.exp(-g))
    h2 = xa_ref.shape[1]
    for i in range(nb):
        o_ref[i, :h2] = xa_ref[i] * gate[i:i + 1]
        o_ref[i, h2:] = xb_ref[i] * gate[i:i + 1]


def kernel(x_nchw, w1, w2):
    N, C, H, W = x_nchw.shape
    HW = H * W
    Cr = w1.shape[0]
    x = jnp.transpose(x_nchw, (0, 2, 3, 1)).reshape(N, HW, C)
    w2t = w2.T

    NB = 4
    out = pl.pallas_call(
        _se_nhwc_kernel,
        out_shape=jax.ShapeDtypeStruct((N, HW, C), x_nchw.dtype),
        grid=(N // NB,),
        in_specs=[
            pl.BlockSpec((NB, HW // 2, C), lambda b: (b, 0, 0)),
            pl.BlockSpec((NB, HW // 2, C), lambda b: (b, 1, 0)),
            pl.BlockSpec((Cr, C), lambda b: (0, 0)),
            pl.BlockSpec((Cr, C), lambda b: (0, 0)),
        ],
        out_specs=pl.BlockSpec((NB, HW, C), lambda b: (b, 0, 0)),
        compiler_params=pltpu.CompilerParams(
            dimension_semantics=("parallel",),
            vmem_limit_bytes=64 * 1024 * 1024,
        ),
    )(x, x, w1, w2t)

    return out.reshape(N, H, W, C).transpose(0, 3, 1, 2)


# speedup vs baseline: 1.0034x; 1.0034x over previous
import jax
import jax.numpy as jnp
from jax import lax
from jax.experimental import pallas as pl
from jax.experimental.pallas import tpu as pltpu

_INV_SQRT2 = 0.7071067811865476

_ERF_A = (0.254829592, -0.284496736, 1.421413741, -1.453152027, 1.061405429)
_ERF_P = 0.3275911


def _erf_approx(v):
    a1, a2, a3, a4, a5 = _ERF_A
    s = jnp.where(v < 0.0, -1.0, 1.0)
    av = jnp.abs(v)
    t = 1.0 / (1.0 + _ERF_P * av)
    poly = t * (a1 + t * (a2 + t * (a3 + t * (a4 + t * a5))))
    return s * (1.0 - poly * jnp.exp(-av * av))


def _gelu(v):
    return 0.5 * v * (1.0 + _erf_approx(v * _INV_SQRT2))


def _se_nhwc_kernel(x_ref, w1_ref, w2t_ref, o_ref):
    nb = x_ref.shape[0]
    hw = x_ref.shape[1]
    pooled = jnp.concatenate(
        [jnp.sum(x_ref[i], axis=0, keepdims=True) for i in range(nb)], axis=0
    ) * (1.0 / hw)
    h = lax.dot_general(pooled, w1_ref[...],
                        (((1,), (1,)), ((), ())),
                        preferred_element_type=jnp.float32)
    h = _gelu(h)
    g = lax.dot_general(h, w2t_ref[...],
                        (((1,), (0,)), ((), ())),
                        preferred_element_type=jnp.float32)
    gate = 1.0 / (1.0 + jnp.exp(-g))
    for i in range(nb):
        o_ref[i] = x_ref[i] * gate[i:i + 1]


def kernel(x_nchw, w1, w2):
    N, C, H, W = x_nchw.shape
    HW = H * W
    Cr = w1.shape[0]
    x = jnp.transpose(x_nchw, (0, 2, 3, 1)).reshape(N, HW, C)
    w2t = w2.T

    NB = 4
    out = pl.pallas_call(
        _se_nhwc_kernel,
        out_shape=jax.ShapeDtypeStruct((N, HW, C), x_nchw.dtype),
        grid=(N // NB,),
        in_specs=[
            pl.BlockSpec((NB, HW, C), lambda b: (b, 0, 0)),
            pl.BlockSpec((Cr, C), lambda b: (0, 0)),
            pl.BlockSpec((Cr, C), lambda b: (0, 0)),
        ],
        out_specs=pl.BlockSpec((NB, HW, C), lambda b: (b, 0, 0)),
        compiler_params=pltpu.CompilerParams(
            dimension_semantics=("parallel",),
            vmem_limit_bytes=64 * 1024 * 1024,
        ),
    )(x, w1, w2t)

    return out.reshape(N, H, W, C).transpose(0, 3, 1, 2)
